# baseline (speedup 1.0000x reference)
import functools

import jax
import jax.numpy as jnp
from jax import lax
from jax.experimental import pallas as pl
from jax.experimental.pallas import tpu as pltpu
from jax.experimental.pallas import tpu_sc as plsc

_C = 55
_EPS = 1e-05
_CPAD = 64
_LANES = 16
_NC = 2
_NS = 16
_NW = _NC * _NS
_REGW = _CPAD * _LANES
_HISTW = 2 * _REGW
_UNROLL = 8
_BANKS = 4


def _sc_histograms(inp_flat, tgt_flat):
    n = inp_flat.shape[0]
    chunk = n // _NW
    iters = chunk // _LANES
    mesh = plsc.VectorSubcoreMesh(core_axis_name="c", subcore_axis_name="s")

    @functools.partial(
        pl.kernel,
        mesh=mesh,
        out_type=jax.ShapeDtypeStruct((_NW, _HISTW), jnp.int32),
        scratch_types=[
            pltpu.VMEM((chunk,), jnp.float32),
            pltpu.VMEM((chunk,), jnp.int32),
            pltpu.VMEM((_HISTW,), jnp.int32),
            pltpu.SemaphoreType.DMA,
            pltpu.SemaphoreType.DMA,
            pltpu.SemaphoreType.DMA,
            pltpu.SemaphoreType.DMA,
        ],
        compiler_params=pltpu.CompilerParams(needs_layout_passes=False),
    )
    def hist_kernel(inp_hbm, tgt_hbm, out_hbm, inp_v, tgt_v, hist_v,
                    sem_a0, sem_b0, sem_a1, sem_b1):
        half = chunk // 2
        wid = lax.axis_index("s") * _NC + lax.axis_index("c")
        base = wid * chunk
        cp_a0 = pltpu.async_copy(
            inp_hbm.at[pl.ds(base, half)], inp_v.at[pl.ds(0, half)], sem_a0)
        cp_b0 = pltpu.async_copy(
            tgt_hbm.at[pl.ds(base, half)], tgt_v.at[pl.ds(0, half)], sem_b0)
        cp_a1 = pltpu.async_copy(
            inp_hbm.at[pl.ds(base + half, half)],
            inp_v.at[pl.ds(half, half)], sem_a1)
        cp_b1 = pltpu.async_copy(
            tgt_hbm.at[pl.ds(base + half, half)],
            tgt_v.at[pl.ds(half, half)], sem_b1)

        zeros_i = jnp.zeros((_LANES,), jnp.int32)
        ones_i = jnp.ones((_LANES,), jnp.int32)
        pk_eq = jnp.full((_LANES,), 65537, jnp.int32)
        pk_ne = jnp.full((_LANES,), 65536, jnp.int32)
        lane = lax.iota(jnp.int32, _LANES)
        lane_b = lane + _REGW

        @plsc.parallel_loop(0, _HISTW // _LANES, 1, unroll=8)
        def _zero(i):
            hist_v[pl.ds(i * _LANES, _LANES)] = zeros_i

        def step(j):
            av = inp_v[pl.ds(j * _LANES, _LANES)]
            bv = tgt_v[pl.ds(j * _LANES, _LANES)]
            ia = (av + 0.5).astype(jnp.int32)
            fa = ia * _LANES + lane
            fb = bv * _LANES + lane_b
            vala = jnp.where(ia == bv, pk_eq, pk_ne)
            plsc.addupdate_scatter(hist_v, [fa], vala)
            plsc.addupdate_scatter(hist_v, [fb], ones_i)

        cp_a0.wait()
        cp_b0.wait()

        @plsc.parallel_loop(0, iters // 2, 1, unroll=_UNROLL)
        def _loop0(j):
            step(j)

        cp_a1.wait()
        cp_b1.wait()

        @plsc.parallel_loop(iters // 2, iters, 1, unroll=_UNROLL)
        def _loop1(j):
            step(j)

        pltpu.sync_copy(hist_v, out_hbm.at[wid])

    return hist_kernel(inp_flat, tgt_flat)


def _finish_body(h_ref, o_ref):
    h = h_ref[:]
    pa = h[:, :_REGW]
    hb = h[:, _REGW:].astype(jnp.float32)
    ha = jnp.right_shift(pa, 16).astype(jnp.float32)
    hj = jnp.bitwise_and(pa, 65535).astype(jnp.float32)
    row = lax.broadcasted_iota(jnp.int32, (_REGW, _CPAD), 0) // _LANES
    col = lax.broadcasted_iota(jnp.int32, (_REGW, _CPAD), 1)
    sel = (row == col).astype(jnp.float32)
    sa = jnp.sum(ha, axis=0, keepdims=True)
    sb = jnp.sum(hb, axis=0, keepdims=True)
    sj = jnp.sum(hj, axis=0, keepdims=True)
    ra = jnp.dot(sa, sel, preferred_element_type=jnp.float32)
    rb = jnp.dot(sb, sel, preferred_element_type=jnp.float32)
    rj = jnp.dot(sj, sel, preferred_element_type=jnp.float32)
    den = ra + rb
    dice = 2.0 * rj / jnp.maximum(den, _EPS)
    ch = lax.broadcasted_iota(jnp.int32, (1, _CPAD), 1)
    valid = jnp.logical_and(ch >= 1, ch <= _C - 1)
    loss = jnp.where(valid, 1.0 - dice, 0.0)
    o_ref[...] = jnp.sum(loss, axis=(0, 1), keepdims=True) / (_C - 1)


def kernel(input, target):
    inp_flat = input.reshape(-1)
    tgt_flat = target.reshape(-1)
    hists = _sc_histograms(inp_flat, tgt_flat)
    out = pl.pallas_call(
        _finish_body,
        out_shape=jax.ShapeDtypeStruct((1, 1), jnp.float32),
    )(hists)
    return out[0, 0]


# speedup vs baseline: 1.0101x; 1.0101x over previous
import functools

import jax
import jax.numpy as jnp
from jax import lax
from jax.experimental import pallas as pl
from jax.experimental.pallas import tpu as pltpu
from jax.experimental.pallas import tpu_sc as plsc

_C = 55
_EPS = 1e-05
_CPAD = 64
_LANES = 16
_NC = 2
_NS = 16
_NW = _NC * _NS
_REGW = _CPAD * _LANES
_HISTW = 2 * _REGW
_UNROLL = 8
_BANKS = 4


def _sc_histograms(inp_flat, tgt_flat):
    n = inp_flat.shape[0]
    chunk = n // _NW
    iters = chunk // _LANES
    mesh = plsc.VectorSubcoreMesh(core_axis_name="c", subcore_axis_name="s")

    @functools.partial(
        pl.kernel,
        mesh=mesh,
        out_type=jax.ShapeDtypeStruct((_NW, _HISTW), jnp.int32),
        scratch_types=[
            pltpu.VMEM((chunk,), jnp.float32),
            pltpu.VMEM((chunk,), jnp.int32),
            pltpu.VMEM((_HISTW,), jnp.int32),
            pltpu.SemaphoreType.DMA,
            pltpu.SemaphoreType.DMA,
        ],
        compiler_params=pltpu.CompilerParams(needs_layout_passes=False),
    )
    def hist_kernel(inp_hbm, tgt_hbm, out_hbm, inp_v, tgt_v, hist_v,
                    sem_a, sem_b):
        wid = lax.axis_index("s") * _NC + lax.axis_index("c")
        base = wid * chunk
        cp_a = pltpu.async_copy(inp_hbm.at[pl.ds(base, chunk)], inp_v, sem_a)
        cp_b = pltpu.async_copy(tgt_hbm.at[pl.ds(base, chunk)], tgt_v, sem_b)

        zeros_i = jnp.zeros((_LANES,), jnp.int32)
        ones_i = jnp.ones((_LANES,), jnp.int32)
        pk_eq = jnp.full((_LANES,), 65537, jnp.int32)
        pk_ne = jnp.full((_LANES,), 65536, jnp.int32)
        lane = lax.iota(jnp.int32, _LANES)
        lane_b = lane + _REGW

        with jax.named_scope("zero_and_stage"):
            @plsc.parallel_loop(0, _HISTW // _LANES, 1, unroll=8)
            def _zero(i):
                hist_v[pl.ds(i * _LANES, _LANES)] = zeros_i

            cp_a.wait()
            cp_b.wait()

        with jax.named_scope("histo_loop"):
            @plsc.parallel_loop(0, iters, 1, unroll=_UNROLL)
            def _loop(j):
                av = inp_v[pl.ds(j * _LANES, _LANES)]
                bv = tgt_v[pl.ds(j * _LANES, _LANES)]
                ia = (av + 0.5).astype(jnp.int32)
                fa = ia * _LANES + lane
                fb = bv * _LANES + lane_b
                vala = jnp.where(ia == bv, pk_eq, pk_ne)
                plsc.addupdate_scatter(hist_v, [fa], vala)
                plsc.addupdate_scatter(hist_v, [fb], ones_i)

        with jax.named_scope("writeback"):
            pltpu.sync_copy(hist_v, out_hbm.at[wid])

    return hist_kernel(inp_flat, tgt_flat)


def _finish_body(h_ref, o_ref):
    h = h_ref[:]
    pa = h[:, :_REGW]
    hb = h[:, _REGW:].astype(jnp.float32)
    ha = jnp.right_shift(pa, 16).astype(jnp.float32)
    hj = jnp.bitwise_and(pa, 65535).astype(jnp.float32)
    row = lax.broadcasted_iota(jnp.int32, (_REGW, _CPAD), 0) // _LANES
    col = lax.broadcasted_iota(jnp.int32, (_REGW, _CPAD), 1)
    sel = (row == col).astype(jnp.float32)
    sa = jnp.sum(ha, axis=0, keepdims=True)
    sb = jnp.sum(hb, axis=0, keepdims=True)
    sj = jnp.sum(hj, axis=0, keepdims=True)
    ra = jnp.dot(sa, sel, preferred_element_type=jnp.float32)
    rb = jnp.dot(sb, sel, preferred_element_type=jnp.float32)
    rj = jnp.dot(sj, sel, preferred_element_type=jnp.float32)
    den = ra + rb
    dice = 2.0 * rj / jnp.maximum(den, _EPS)
    ch = lax.broadcasted_iota(jnp.int32, (1, _CPAD), 1)
    valid = jnp.logical_and(ch >= 1, ch <= _C - 1)
    loss = jnp.where(valid, 1.0 - dice, 0.0)
    o_ref[...] = jnp.sum(loss, axis=(0, 1), keepdims=True) / (_C - 1)


def kernel(input, target):
    inp_flat = input.reshape(-1)
    tgt_flat = target.reshape(-1)
    hists = _sc_histograms(inp_flat, tgt_flat)
    out = pl.pallas_call(
        _finish_body,
        out_shape=jax.ShapeDtypeStruct((1, 1), jnp.float32),
    )(hists)
    return out[0, 0]


# speedup vs baseline: 1.0155x; 1.0054x over previous
import functools

import jax
import jax.numpy as jnp
from jax import lax
from jax.experimental import pallas as pl
from jax.experimental.pallas import tpu as pltpu
from jax.experimental.pallas import tpu_sc as plsc

_C = 55
_EPS = 1e-05
_CPAD = 64
_LANES = 16
_NC = 2
_NS = 16
_NW = _NC * _NS
_REGW = _CPAD * _LANES
_HISTW = 2 * _REGW
_UNROLL = 8
_BANKS = 4


def _sc_histograms(inp_flat, tgt_flat):
    n = inp_flat.shape[0]
    chunk = n // _NW
    iters = chunk // _LANES
    mesh = plsc.VectorSubcoreMesh(core_axis_name="c", subcore_axis_name="s")

    @functools.partial(
        pl.kernel,
        mesh=mesh,
        out_type=jax.ShapeDtypeStruct((_NW, _HISTW), jnp.int32),
        scratch_types=[
            pltpu.VMEM((chunk,), jnp.float32),
            pltpu.VMEM((chunk,), jnp.int32),
            pltpu.VMEM((_HISTW,), jnp.int32),
            pltpu.SemaphoreType.DMA,
            pltpu.SemaphoreType.DMA,
        ],
        compiler_params=pltpu.CompilerParams(needs_layout_passes=False),
    )
    def hist_kernel(inp_hbm, tgt_hbm, out_hbm, inp_v, tgt_v, hist_v,
                    sem_a, sem_b):
        wid = lax.axis_index("s") * _NC + lax.axis_index("c")
        base = wid * chunk
        cp_a = pltpu.async_copy(inp_hbm.at[pl.ds(base, chunk)], inp_v, sem_a)
        cp_b = pltpu.async_copy(tgt_hbm.at[pl.ds(base, chunk)], tgt_v, sem_b)

        zeros_i = jnp.zeros((_LANES,), jnp.int32)
        ones_i = jnp.ones((_LANES,), jnp.int32)
        pk_eq = jnp.full((_LANES,), 65537, jnp.int32)
        pk_ne = jnp.full((_LANES,), 65536, jnp.int32)
        lane = lax.iota(jnp.int32, _LANES)
        lane_b = lane + _REGW

        with jax.named_scope("zero_and_stage"):
            @plsc.parallel_loop(0, _HISTW // _LANES, 1, unroll=8)
            def _zero(i):
                hist_v[pl.ds(i * _LANES, _LANES)] = zeros_i

            cp_a.wait()
            cp_b.wait()

        with jax.named_scope("histo_loop"):
            @plsc.parallel_loop(0, iters, 1, unroll=_UNROLL)
            def _loop(j):
                av = inp_v[pl.ds(j * _LANES, _LANES)]
                bv = tgt_v[pl.ds(j * _LANES, _LANES)]
                ia = (av + 0.5).astype(jnp.int32)
                fa = ia * _LANES + lane
                fb = bv * _LANES + lane_b
                vala = jnp.where(ia == bv, pk_eq, pk_ne)
                plsc.addupdate_scatter(hist_v, [fa], vala)
                plsc.addupdate_scatter(hist_v, [fb], ones_i)

        with jax.named_scope("writeback"):
            pltpu.sync_copy(hist_v, out_hbm.at[wid])

    return hist_kernel(inp_flat, tgt_flat)


def _finish_body(h_ref, o_ref):
    h = h_ref[:]
    pa = h[:, :_REGW]
    hb = h[:, _REGW:].astype(jnp.float32)
    ha = jnp.right_shift(pa, 16).astype(jnp.float32)
    hj = jnp.bitwise_and(pa, 65535).astype(jnp.float32)
    row = lax.broadcasted_iota(jnp.int32, (_REGW, _CPAD), 0) // _LANES
    col = lax.broadcasted_iota(jnp.int32, (_REGW, _CPAD), 1)
    sel = (row == col).astype(jnp.float32)
    sa = jnp.sum(ha, axis=0, keepdims=True)
    sb = jnp.sum(hb, axis=0, keepdims=True)
    sj = jnp.sum(hj, axis=0, keepdims=True)
    s3 = jnp.concatenate([sa, sb, sj], axis=0)
    r3 = jnp.dot(s3, sel, preferred_element_type=jnp.float32)
    den = r3[0:1] + r3[1:2]
    dice = 2.0 * r3[2:3] / jnp.maximum(den, _EPS)
    ch = lax.broadcasted_iota(jnp.int32, (1, _CPAD), 1)
    valid = jnp.logical_and(ch >= 1, ch <= _C - 1)
    loss = jnp.where(valid, 1.0 - dice, 0.0)
    o_ref[...] = jnp.sum(loss, axis=(0, 1), keepdims=True) / (_C - 1)


def kernel(input, target):
    inp_flat = input.reshape(-1)
    tgt_flat = target.reshape(-1)
    hists = _sc_histograms(inp_flat, tgt_flat)
    out = pl.pallas_call(
        _finish_body,
        out_shape=jax.ShapeDtypeStruct((1, 1), jnp.float32),
    )(hists)
    return out[0, 0]
